# baseline (device time: 300157 ns/iter reference)
import jax
import jax.numpy as jnp
from jax import lax
from jax.experimental import pallas as pl
from jax.experimental.pallas import tpu as pltpu


def _allreduce_y(partial):
    m, n = partial.shape

    def body(p_ref, out_ref, recv_ref, send_sem, recv_sem):
        my_x = lax.axis_index("x")
        my_y = lax.axis_index("y")
        peer = (my_x, 1 - my_y)

        barrier_sem = pltpu.get_barrier_semaphore()
        pl.semaphore_signal(
            barrier_sem, inc=1, device_id=peer,
            device_id_type=pl.DeviceIdType.MESH,
        )
        pl.semaphore_wait(barrier_sem, 1)

        rdma = pltpu.make_async_remote_copy(
            src_ref=p_ref,
            dst_ref=recv_ref,
            send_sem=send_sem,
            recv_sem=recv_sem,
            device_id=peer,
            device_id_type=pl.DeviceIdType.MESH,
        )
        rdma.start()
        rdma.wait()
        out_ref[...] = p_ref[...] + recv_ref[...]

    return pl.pallas_call(
        body,
        out_shape=jax.ShapeDtypeStruct((m, n), partial.dtype),
        in_specs=[pl.BlockSpec(memory_space=pltpu.VMEM)],
        out_specs=pl.BlockSpec(memory_space=pltpu.VMEM),
        scratch_shapes=[
            pltpu.VMEM((m, n), partial.dtype),
            pltpu.SemaphoreType.DMA,
            pltpu.SemaphoreType.DMA,
        ],
        compiler_params=pltpu.CompilerParams(collective_id=0),
    )(partial)


def kernel(dy, W):
    partial = lax.dot_general(
        dy, W,
        dimension_numbers=(((1,), (1,)), ((), ())),
        preferred_element_type=jnp.float32,
    )
    return _allreduce_y(partial)


# device time: 143493 ns/iter; 2.0918x vs baseline; 2.0918x over previous
import jax
import jax.numpy as jnp
from jax import lax
from jax.experimental import pallas as pl
from jax.experimental.pallas import tpu as pltpu

M, K, N = 2048, 8192, 2048
C = 16
CN = N // C
S = 4
HALF = M // 2


def kernel(dy, W):
    def body(dy_ref, w_ref, out_ref, dy_v, w_buf, y_recv,
             dy_sem, w_sems, y_send_sems, y_recv_sems,
             x_send_sems, x_recv_sems, credit_sem):
        my_x = lax.axis_index("x")
        my_y = lax.axis_index("y")
        y_peer = (my_x, 1 - my_y)
        x_peer = (1 - my_x, my_y)
        rows = pl.ds(my_x * HALF, HALF)

        def w_copy(c):
            slot = lax.rem(c, 2)
            return pltpu.make_async_copy(
                w_ref.at[pl.ds(c * CN, CN), :],
                w_buf.at[slot], w_sems.at[slot])

        def y_rdma(c):
            cols = pl.ds(c * CN, CN)
            return pltpu.make_async_remote_copy(
                src_ref=out_ref.at[rows, cols],
                dst_ref=y_recv.at[lax.rem(c, S)],
                send_sem=y_send_sems.at[c], recv_sem=y_recv_sems.at[c],
                device_id=y_peer, device_id_type=pl.DeviceIdType.MESH)

        def x_rdma(c):
            cols = pl.ds(c * CN, CN)
            return pltpu.make_async_remote_copy(
                src_ref=out_ref.at[rows, cols],
                dst_ref=out_ref.at[rows, cols],
                send_sem=x_send_sems.at[c], recv_sem=x_recv_sems.at[c],
                device_id=x_peer, device_id_type=pl.DeviceIdType.MESH)

        dy_copy = pltpu.make_async_copy(dy_ref.at[rows, :], dy_v, dy_sem)
        dy_copy.start()
        w_copy(0).start()

        barrier_sem = pltpu.get_barrier_semaphore()
        for peer in (y_peer, x_peer):
            pl.semaphore_signal(
                barrier_sem, inc=1, device_id=peer,
                device_id_type=pl.DeviceIdType.MESH)
        pl.semaphore_wait(barrier_sem, 2)

        dy_copy.wait()

        def consume(d):
            cols = pl.ds(d * CN, CN)
            yr = y_rdma(d)
            yr.wait_recv()
            yr.wait_send()
            out_ref[rows, cols] = out_ref[rows, cols] + y_recv[lax.rem(d, S)]

            @pl.when(d + S < C)
            def _():
                pl.semaphore_signal(
                    credit_sem, inc=1, device_id=y_peer,
                    device_id_type=pl.DeviceIdType.MESH)

            x_rdma(d).start()

        def step(c, carry):
            w_copy(c).wait()

            @pl.when(c + 1 < C)
            def _():
                w_copy(c + 1).start()

            p = lax.dot_general(
                dy_v[...], w_buf[lax.rem(c, 2)],
                dimension_numbers=(((1,), (1,)), ((), ())),
                preferred_element_type=jnp.float32)
            cols = pl.ds(c * CN, CN)
            out_ref[rows, cols] = p

            @pl.when(c >= S)
            def _():
                pl.semaphore_wait(credit_sem, 1)

            y_rdma(c).start()

            @pl.when(c >= 1)
            def _():
                consume(c - 1)

            return carry

        lax.fori_loop(0, C, step, 0)
        consume(C - 1)

        def wait_step(c, carry):
            xr = x_rdma(c)
            xr.wait_send()
            xr.wait_recv()
            return carry

        lax.fori_loop(0, C, wait_step, 0)

    return pl.pallas_call(
        body,
        out_shape=jax.ShapeDtypeStruct((M, N), jnp.float32),
        in_specs=[
            pl.BlockSpec(memory_space=pltpu.MemorySpace.HBM),
            pl.BlockSpec(memory_space=pltpu.MemorySpace.HBM),
        ],
        out_specs=pl.BlockSpec(memory_space=pltpu.VMEM),
        scratch_shapes=[
            pltpu.VMEM((HALF, K), jnp.float32),
            pltpu.VMEM((2, CN, K), jnp.float32),
            pltpu.VMEM((S, HALF, CN), jnp.float32),
            pltpu.SemaphoreType.DMA,
            pltpu.SemaphoreType.DMA((2,)),
            pltpu.SemaphoreType.DMA((C,)),
            pltpu.SemaphoreType.DMA((C,)),
            pltpu.SemaphoreType.DMA((C,)),
            pltpu.SemaphoreType.DMA((C,)),
            pltpu.SemaphoreType.REGULAR,
        ],
        compiler_params=pltpu.CompilerParams(
            collective_id=0,
            vmem_limit_bytes=64 * 1024 * 1024,
        ),
    )(dy, W)


# device time: 139789 ns/iter; 2.1472x vs baseline; 1.0265x over previous
import jax
import jax.numpy as jnp
from jax import lax
from jax.experimental import pallas as pl
from jax.experimental.pallas import tpu as pltpu

M, K, N = 2048, 8192, 2048
C = 16
CN = N // C
S = 4
HALF = M // 2


def kernel(dy, W):
    def body(dy_ref, w_ref, out_ref, dy_v, w_buf, y_recv,
             dy_sem, w_sems, y_send_sems, y_recv_sems,
             x_send_sems, x_recv_sems, credit_sem):
        my_x = lax.axis_index("x")
        my_y = lax.axis_index("y")
        y_peer = (my_x, 1 - my_y)
        x_peer = (1 - my_x, my_y)
        rows = pl.ds(my_x * HALF, HALF)

        def w_copy(c):
            slot = lax.rem(c, 2)
            return pltpu.make_async_copy(
                w_ref.at[pl.ds(c * CN, CN), :],
                w_buf.at[slot], w_sems.at[slot])

        def y_rdma(c):
            cols = pl.ds(c * CN, CN)
            return pltpu.make_async_remote_copy(
                src_ref=out_ref.at[rows, cols],
                dst_ref=y_recv.at[lax.rem(c, S)],
                send_sem=y_send_sems.at[c], recv_sem=y_recv_sems.at[c],
                device_id=y_peer, device_id_type=pl.DeviceIdType.MESH)

        def x_rdma(c):
            cols = pl.ds(c * CN, CN)
            return pltpu.make_async_remote_copy(
                src_ref=out_ref.at[rows, cols],
                dst_ref=out_ref.at[rows, cols],
                send_sem=x_send_sems.at[c], recv_sem=x_recv_sems.at[c],
                device_id=x_peer, device_id_type=pl.DeviceIdType.MESH)

        dy_copy = pltpu.make_async_copy(dy_ref.at[rows, :], dy_v, dy_sem)
        dy_copy.start()
        w_copy(0).start()
        w_copy(1).start()

        barrier_sem = pltpu.get_barrier_semaphore()
        for peer in (y_peer, x_peer):
            pl.semaphore_signal(
                barrier_sem, inc=1, device_id=peer,
                device_id_type=pl.DeviceIdType.MESH)
        pl.semaphore_wait(barrier_sem, 2)

        dy_copy.wait()

        def consume(d):
            cols = pl.ds(d * CN, CN)
            yr = y_rdma(d)
            yr.wait_recv()
            yr.wait_send()
            out_ref[rows, cols] = out_ref[rows, cols] + y_recv[lax.rem(d, S)]

            @pl.when(d + S < C)
            def _():
                pl.semaphore_signal(
                    credit_sem, inc=1, device_id=y_peer,
                    device_id_type=pl.DeviceIdType.MESH)

            x_rdma(d).start()

        def step(c, carry):
            w_copy(c).wait()

            p = lax.dot_general(
                dy_v[...], w_buf[lax.rem(c, 2)],
                dimension_numbers=(((1,), (1,)), ((), ())),
                preferred_element_type=jnp.float32)

            @pl.when(c + 2 < C)
            def _():
                w_copy(c + 2).start()

            cols = pl.ds(c * CN, CN)
            out_ref[rows, cols] = p

            @pl.when(c >= S)
            def _():
                pl.semaphore_wait(credit_sem, 1)

            y_rdma(c).start()

            @pl.when(c >= 2)
            def _():
                consume(c - 2)

            return carry

        lax.fori_loop(0, C, step, 0)
        consume(C - 2)
        consume(C - 1)

        def wait_step(c, carry):
            xr = x_rdma(c)
            xr.wait_send()
            xr.wait_recv()
            return carry

        lax.fori_loop(0, C, wait_step, 0)

    return pl.pallas_call(
        body,
        out_shape=jax.ShapeDtypeStruct((M, N), jnp.float32),
        in_specs=[
            pl.BlockSpec(memory_space=pltpu.MemorySpace.HBM),
            pl.BlockSpec(memory_space=pltpu.MemorySpace.HBM),
        ],
        out_specs=pl.BlockSpec(memory_space=pltpu.VMEM),
        scratch_shapes=[
            pltpu.VMEM((HALF, K), jnp.float32),
            pltpu.VMEM((2, CN, K), jnp.float32),
            pltpu.VMEM((S, HALF, CN), jnp.float32),
            pltpu.SemaphoreType.DMA,
            pltpu.SemaphoreType.DMA((2,)),
            pltpu.SemaphoreType.DMA((C,)),
            pltpu.SemaphoreType.DMA((C,)),
            pltpu.SemaphoreType.DMA((C,)),
            pltpu.SemaphoreType.DMA((C,)),
            pltpu.SemaphoreType.REGULAR,
        ],
        compiler_params=pltpu.CompilerParams(
            collective_id=0,
            vmem_limit_bytes=64 * 1024 * 1024,
        ),
    )(dy, W)
